# Initial kernel scaffold; baseline (speedup 1.0000x reference)
import functools

import jax
import jax.numpy as jnp
from jax import lax
from jax.experimental import pallas as pl
from jax.experimental.pallas import tpu as pltpu
from jax.experimental.pallas import tpu_sc as plsc

N = 10000
D = 128
E = 320000

NC = 2
NS = 16
NW = NC * NS
C = 128
CH = 80
EPW = C * CH
EPAD = EPW * NW
NPAD = 10240
RPT = NPAD // NS


def _make_agg(with_cnt: bool):
    mesh = plsc.VectorSubcoreMesh(core_axis_name="c", subcore_axis_name="s")
    out_type = [jax.ShapeDtypeStruct((NC, NPAD, D), jnp.float32)]
    scratch = [
        pltpu.VMEM((CH, C), jnp.int32),
        pltpu.VMEM((CH, C), jnp.int32),
        pltpu.VMEM((C, D), jnp.float32),
        pltpu.VMEM_SHARED((NPAD, D), jnp.float32),
        pltpu.SemaphoreType.DMA,
    ]
    if with_cnt:
        out_type.append(jax.ShapeDtypeStruct((NC, NPAD), jnp.float32))
        scratch += [
            pltpu.VMEM((C,), jnp.float32),
            pltpu.VMEM((RPT,), jnp.float32),
            pltpu.VMEM_SHARED((NPAD,), jnp.float32),
        ]

    def body(h_hbm, srcr_hbm, dstr_hbm, out_hbm, *rest):
        if with_cnt:
            (cnt_hbm, src_v, dst_v, rows_v, accum_sh, sem,
             ones_v, zc_v, cnt_sh) = rest
        else:
            (src_v, dst_v, rows_v, accum_sh, sem) = rest
        cid = lax.axis_index("c")
        sid = lax.axis_index("s")
        w = sid * NC + cid

        pltpu.sync_copy(srcr_hbm.at[w], src_v)
        pltpu.sync_copy(dstr_hbm.at[w], dst_v)

        z16 = jnp.zeros((16,), jnp.float32)

        def zrow(i, carry):
            for j in range(D // 16):
                rows_v[i, pl.ds(j * 16, 16)] = z16
            return carry

        lax.fori_loop(0, C, zrow, 0)
        for k in range(RPT // C):
            pltpu.sync_copy(rows_v, accum_sh.at[pl.ds(sid * RPT + k * C, C)])

        if with_cnt:
            one16 = jnp.ones((16,), jnp.float32)

            def fill(i, carry):
                ones_v[pl.ds(i * 16, 16)] = one16
                return carry

            lax.fori_loop(0, C // 16, fill, 0)

            def zfill(i, carry):
                zc_v[pl.ds(i * 16, 16)] = z16
                return carry

            lax.fori_loop(0, RPT // 16, zfill, 0)
            pltpu.sync_copy(zc_v, cnt_sh.at[pl.ds(sid * RPT, RPT)])

        plsc.subcore_barrier()

        def chunk(i, carry):
            pltpu.async_copy(h_hbm.at[src_v.at[i]], rows_v, sem).wait()
            pltpu.sync_copy(rows_v, accum_sh.at[dst_v.at[i]], add=True)
            if with_cnt:
                pltpu.sync_copy(ones_v, cnt_sh.at[dst_v.at[i]], add=True)
            return carry

        lax.fori_loop(0, CH, chunk, 0)
        plsc.subcore_barrier()

        pltpu.sync_copy(accum_sh.at[pl.ds(sid * RPT, RPT)],
                        out_hbm.at[cid].at[pl.ds(sid * RPT, RPT)])
        if with_cnt:
            pltpu.sync_copy(cnt_sh.at[pl.ds(sid * RPT, RPT)],
                            cnt_hbm.at[cid].at[pl.ds(sid * RPT, RPT)])

    return pl.kernel(body, out_type=out_type, mesh=mesh, scratch_types=scratch)


_agg_cnt = _make_agg(True)
_agg = _make_agg(False)

BR = 1024


def _dense_body(final, p_ref, cnt_ref, h_ref, wn_ref, ws_ref, b_ref, o_ref):
    c = cnt_ref[0] + cnt_ref[1]
    inv = 1.0 / jnp.maximum(c, 1.0)
    mean = (p_ref[0] + p_ref[1]) * inv
    y = (jnp.dot(mean, wn_ref[...], preferred_element_type=jnp.float32)
         + jnp.dot(h_ref[...], ws_ref[...], preferred_element_type=jnp.float32)
         + b_ref[...])
    z = jnp.maximum(y, 0.0)
    if final:
        m = jnp.max(z, axis=1, keepdims=True)
        e = jnp.exp(z - m)
        s = jnp.sum(e, axis=1, keepdims=True)
        z = z - m - jnp.log(s)
    o_ref[...] = z


def _dense(p, cnt3, h, Wn, Ws, b, final):
    return pl.pallas_call(
        functools.partial(_dense_body, final),
        grid=(NPAD // BR,),
        in_specs=[
            pl.BlockSpec((NC, BR, D), lambda i: (0, i, 0)),
            pl.BlockSpec((NC, BR, 1), lambda i: (0, i, 0)),
            pl.BlockSpec((BR, D), lambda i: (i, 0)),
            pl.BlockSpec((D, D), lambda i: (0, 0)),
            pl.BlockSpec((D, D), lambda i: (0, 0)),
            pl.BlockSpec((1, D), lambda i: (0, 0)),
        ],
        out_specs=pl.BlockSpec((BR, D), lambda i: (i, 0)),
        out_shape=jax.ShapeDtypeStruct((NPAD, D), jnp.float32),
    )(p, cnt3, h, Wn, Ws, b.reshape(1, D))


def kernel(x, edge_index, W1n, W1s, b1, W2n, W2s, b2, W3n, W3s, b3):
    src = edge_index[0]
    dst = edge_index[1]
    pad = EPAD - E
    src_p = jnp.concatenate([src, jnp.zeros((pad,), jnp.int32)])
    dst_p = jnp.concatenate([dst, jnp.full((pad,), N, jnp.int32)])
    srcr = src_p.reshape(NW, CH, C)
    dstr = dst_p.reshape(NW, CH, C)
    x_pad = jnp.concatenate(
        [x, jnp.zeros((NPAD - N, D), jnp.float32)], axis=0)

    p1, cnt = _agg_cnt(x_pad, srcr, dstr)
    cnt3 = cnt[:, :, None]
    h1 = _dense(p1, cnt3, x_pad, W1n, W1s, b1, final=False)
    p2 = _agg(h1, srcr, dstr)
    h2 = _dense(p2, cnt3, h1, W2n, W2s, b2, final=False)
    p3 = _agg(h2, srcr, dstr)
    out = _dense(p3, cnt3, h2, W3n, W3s, b3, final=True)
    return out[:N]


# speedup vs baseline: 3.4224x; 3.4224x over previous
import functools

import jax
import jax.numpy as jnp
from jax import lax
from jax.experimental import pallas as pl
from jax.experimental.pallas import tpu as pltpu
from jax.experimental.pallas import tpu_sc as plsc

N = 10000
D = 128
E = 320000

NC = 2
NS = 16
NW = NC * NS
C = 128
CH = 80
EPW = C * CH
EPAD = EPW * NW
NPAD = 10240
RPT = NPAD // NS


def _make_agg(with_cnt: bool):
    mesh = plsc.VectorSubcoreMesh(core_axis_name="c", subcore_axis_name="s")
    out_type = [jax.ShapeDtypeStruct((NC, NPAD, D), jnp.float32)]
    scratch = [
        pltpu.VMEM((CH, C), jnp.int32),
        pltpu.VMEM((CH, C), jnp.int32),
        pltpu.VMEM((C, D), jnp.float32),
        pltpu.VMEM_SHARED((NPAD, D), jnp.float32),
        pltpu.SemaphoreType.DMA,
    ]
    if with_cnt:
        out_type.append(jax.ShapeDtypeStruct((NC, NPAD), jnp.float32))
        scratch += [
            pltpu.VMEM((C,), jnp.float32),
            pltpu.VMEM((RPT,), jnp.float32),
            pltpu.VMEM_SHARED((NPAD,), jnp.float32),
        ]

    def body(h_hbm, srcr_hbm, dstr_hbm, out_hbm, *rest):
        if with_cnt:
            (cnt_hbm, src_v, dst_v, rows_v, accum_sh, sem,
             ones_v, zc_v, cnt_sh) = rest
        else:
            (src_v, dst_v, rows_v, accum_sh, sem) = rest
        cid = lax.axis_index("c")
        sid = lax.axis_index("s")
        w = sid * NC + cid

        pltpu.sync_copy(srcr_hbm.at[w], src_v)
        pltpu.sync_copy(dstr_hbm.at[w], dst_v)

        z16 = jnp.zeros((16,), jnp.float32)

        def zrow(i, carry):
            for j in range(D // 16):
                rows_v[i, pl.ds(j * 16, 16)] = z16
            return carry

        lax.fori_loop(0, C, zrow, 0)
        for k in range(RPT // C):
            pltpu.sync_copy(rows_v, accum_sh.at[pl.ds(sid * RPT + k * C, C)])

        if with_cnt:
            one16 = jnp.ones((16,), jnp.float32)

            def fill(i, carry):
                ones_v[pl.ds(i * 16, 16)] = one16
                return carry

            lax.fori_loop(0, C // 16, fill, 0)

            def zfill(i, carry):
                zc_v[pl.ds(i * 16, 16)] = z16
                return carry

            lax.fori_loop(0, RPT // 16, zfill, 0)
            pltpu.sync_copy(zc_v, cnt_sh.at[pl.ds(sid * RPT, RPT)])

        plsc.subcore_barrier()

        def chunk(i, carry):
            pltpu.async_copy(h_hbm.at[src_v.at[i]], rows_v, sem).wait()
            pltpu.sync_copy(rows_v, accum_sh.at[dst_v.at[i]], add=True)
            if with_cnt:
                pltpu.sync_copy(ones_v, cnt_sh.at[dst_v.at[i]], add=True)
            return carry

        lax.fori_loop(0, CH, chunk, 0)
        plsc.subcore_barrier()

        pltpu.sync_copy(accum_sh.at[pl.ds(sid * RPT, RPT)],
                        out_hbm.at[cid].at[pl.ds(sid * RPT, RPT)])
        if with_cnt:
            pltpu.sync_copy(cnt_sh.at[pl.ds(sid * RPT, RPT)],
                            cnt_hbm.at[cid].at[pl.ds(sid * RPT, RPT)])

    return pl.kernel(body, out_type=out_type, mesh=mesh, scratch_types=scratch)


_agg_cnt = _make_agg(True)
_agg = _make_agg(False)

BR = 1024


def _dense_body(final, p_ref, cnt_ref, h_ref, wn_ref, ws_ref, b_ref, o_ref):
    c = cnt_ref[0] + cnt_ref[1]
    inv = 1.0 / jnp.maximum(c, 1.0)
    mean = (p_ref[0] + p_ref[1]) * inv
    y = (jnp.dot(mean, wn_ref[...], preferred_element_type=jnp.float32)
         + jnp.dot(h_ref[...], ws_ref[...], preferred_element_type=jnp.float32)
         + b_ref[...])
    z = jnp.maximum(y, 0.0)
    if final:
        m = jnp.max(z, axis=1, keepdims=True)
        e = jnp.exp(z - m)
        s = jnp.sum(e, axis=1, keepdims=True)
        z = z - m - jnp.log(s)
    o_ref[...] = z


def _dense(p, cnt3, h, Wn, Ws, b, final):
    return pl.pallas_call(
        functools.partial(_dense_body, final),
        grid=(NPAD // BR,),
        in_specs=[
            pl.BlockSpec((NC, BR, D), lambda i: (0, i, 0)),
            pl.BlockSpec((NC, BR, 1), lambda i: (0, i, 0)),
            pl.BlockSpec((BR, D), lambda i: (i, 0)),
            pl.BlockSpec((D, D), lambda i: (0, 0)),
            pl.BlockSpec((D, D), lambda i: (0, 0)),
            pl.BlockSpec((1, D), lambda i: (0, 0)),
        ],
        out_specs=pl.BlockSpec((BR, D), lambda i: (i, 0)),
        out_shape=jax.ShapeDtypeStruct((NPAD, D), jnp.float32),
    )(p, cnt3, h, Wn, Ws, b.reshape(1, D))


def kernel(x, edge_index, W1n, W1s, b1, W2n, W2s, b2, W3n, W3s, b3):
    src = edge_index[0]
    dst = edge_index[1]
    pad = EPAD - E
    src_p = jnp.concatenate([src, jnp.zeros((pad,), jnp.int32)])
    dst_p = jnp.concatenate([dst, jnp.full((pad,), N, jnp.int32)])
    srcr = src_p.reshape(NW, CH, C)
    dstr = dst_p.reshape(NW, CH, C)
    x_pad = jnp.concatenate(
        [x, jnp.zeros((NPAD - N, D), jnp.float32)], axis=0)

    p1, cnt = _agg_cnt(x_pad, srcr, dstr)
    cnt3 = cnt[:, :, None]
    h1 = _dense(p1, cnt3, x_pad, W1n, W1s, b1, final=False)
    [p2] = _agg(h1, srcr, dstr)
    h2 = _dense(p2, cnt3, h1, W2n, W2s, b2, final=False)
    [p3] = _agg(h2, srcr, dstr)
    out = _dense(p3, cnt3, h2, W3n, W3s, b3, final=True)
    return out[:N]


# speedup vs baseline: 3.9082x; 1.1419x over previous
import functools

import jax
import jax.numpy as jnp
from jax import lax
from jax.experimental import pallas as pl
from jax.experimental.pallas import tpu as pltpu
from jax.experimental.pallas import tpu_sc as plsc

N = 10000
D = 128
E = 320000

NC = 2
NS = 16
NW = NC * NS
C = 128
CH = 80
NB = 2
EPW = C * CH
EPAD = EPW * NW
NPAD = 10240
RPT = NPAD // NS


def _make_agg(with_cnt: bool):
    mesh = plsc.VectorSubcoreMesh(core_axis_name="c", subcore_axis_name="s")
    out_type = [jax.ShapeDtypeStruct((NC, NPAD, D), jnp.float32)]
    scratch = [
        pltpu.VMEM((NB, C), jnp.int32),
        pltpu.VMEM((NB, C), jnp.int32),
        pltpu.VMEM((NB, C, D), jnp.float32),
        pltpu.VMEM_SHARED((NPAD, D), jnp.float32),
    ] + [pltpu.SemaphoreType.DMA] * (3 * NB)
    if with_cnt:
        out_type.append(jax.ShapeDtypeStruct((NC, NPAD), jnp.float32))
        scratch += [
            pltpu.VMEM((C,), jnp.float32),
            pltpu.VMEM((RPT,), jnp.float32),
            pltpu.VMEM_SHARED((NPAD,), jnp.float32),
        ]

    def body(h_hbm, srcr_hbm, dstr_hbm, out_hbm, *rest):
        if with_cnt:
            (cnt_hbm, src_v, dst_v, rows_v, accum_sh, *sems,
             ones_v, zc_v, cnt_sh) = rest
        else:
            (src_v, dst_v, rows_v, accum_sh, *sems) = rest
        cid = lax.axis_index("c")
        sid = lax.axis_index("s")
        w = sid * NC + cid
        my_src = srcr_hbm.at[w]
        my_dst = dstr_hbm.at[w]

        z16 = jnp.zeros((16,), jnp.float32)

        def zrow(i, carry):
            for j in range(D // 16):
                rows_v[0, i, pl.ds(j * 16, 16)] = z16
            return carry

        lax.fori_loop(0, C, zrow, 0)
        for k in range(RPT // C):
            pltpu.sync_copy(rows_v.at[0],
                            accum_sh.at[pl.ds(sid * RPT + k * C, C)])

        if with_cnt:
            one16 = jnp.ones((16,), jnp.float32)

            def fill(i, carry):
                ones_v[pl.ds(i * 16, 16)] = one16
                return carry

            lax.fori_loop(0, C // 16, fill, 0)

            def zfill(i, carry):
                zc_v[pl.ds(i * 16, 16)] = z16
                return carry

            lax.fori_loop(0, RPT // 16, zfill, 0)
            pltpu.sync_copy(zc_v, cnt_sh.at[pl.ds(sid * RPT, RPT)])

        plsc.subcore_barrier()

        semg = sems[0:NB]
        sems_i = sems[NB:2 * NB]
        semd = sems[2 * NB:3 * NB]

        def gather(i, b):
            pltpu.make_async_copy(
                h_hbm.at[src_v.at[b]], rows_v.at[b], semg[b]).start()

        for b in range(NB):
            pltpu.sync_copy(my_src.at[b], src_v.at[b])
            pltpu.sync_copy(my_dst.at[b], dst_v.at[b])
            gather(b, b)

        def group(g, carry):
            for b in range(NB):
                i = g * NB + b
                pltpu.make_async_copy(
                    h_hbm.at[src_v.at[b]], rows_v.at[b], semg[b]).wait()

                @pl.when(i + NB < CH)
                def _():
                    pltpu.make_async_copy(
                        my_src.at[i + NB], src_v.at[b], sems_i[b]).start()

                @pl.when(i >= NB)
                def _():
                    pltpu.make_async_copy(
                        my_dst.at[0], dst_v.at[b], semd[b]).wait()

                pltpu.sync_copy(rows_v.at[b], accum_sh.at[dst_v.at[b]],
                                add=True)
                if with_cnt:
                    pltpu.sync_copy(ones_v, cnt_sh.at[dst_v.at[b]], add=True)

                @pl.when(i + NB < CH)
                def _():
                    pltpu.make_async_copy(
                        my_dst.at[i + NB], dst_v.at[b], semd[b]).start()
                    pltpu.make_async_copy(
                        my_src.at[0], src_v.at[b], sems_i[b]).wait()
                    gather(i + NB, b)
            return carry

        lax.fori_loop(0, CH // NB, group, 0)
        plsc.subcore_barrier()

        pltpu.sync_copy(accum_sh.at[pl.ds(sid * RPT, RPT)],
                        out_hbm.at[cid].at[pl.ds(sid * RPT, RPT)])
        if with_cnt:
            pltpu.sync_copy(cnt_sh.at[pl.ds(sid * RPT, RPT)],
                            cnt_hbm.at[cid].at[pl.ds(sid * RPT, RPT)])

    return pl.kernel(body, out_type=out_type, mesh=mesh, scratch_types=scratch)


_agg_cnt = _make_agg(True)
_agg = _make_agg(False)

BR = 1024


def _dense_body(final, p_ref, cnt_ref, h_ref, wn_ref, ws_ref, b_ref, o_ref):
    c = cnt_ref[0] + cnt_ref[1]
    inv = 1.0 / jnp.maximum(c, 1.0)
    mean = (p_ref[0] + p_ref[1]) * inv
    y = (jnp.dot(mean, wn_ref[...], preferred_element_type=jnp.float32)
         + jnp.dot(h_ref[...], ws_ref[...], preferred_element_type=jnp.float32)
         + b_ref[...])
    z = jnp.maximum(y, 0.0)
    if final:
        m = jnp.max(z, axis=1, keepdims=True)
        e = jnp.exp(z - m)
        s = jnp.sum(e, axis=1, keepdims=True)
        z = z - m - jnp.log(s)
    o_ref[...] = z


def _dense(p, cnt3, h, Wn, Ws, b, final):
    return pl.pallas_call(
        functools.partial(_dense_body, final),
        grid=(NPAD // BR,),
        in_specs=[
            pl.BlockSpec((NC, BR, D), lambda i: (0, i, 0)),
            pl.BlockSpec((NC, BR, 1), lambda i: (0, i, 0)),
            pl.BlockSpec((BR, D), lambda i: (i, 0)),
            pl.BlockSpec((D, D), lambda i: (0, 0)),
            pl.BlockSpec((D, D), lambda i: (0, 0)),
            pl.BlockSpec((1, D), lambda i: (0, 0)),
        ],
        out_specs=pl.BlockSpec((BR, D), lambda i: (i, 0)),
        out_shape=jax.ShapeDtypeStruct((NPAD, D), jnp.float32),
    )(p, cnt3, h, Wn, Ws, b.reshape(1, D))


def kernel(x, edge_index, W1n, W1s, b1, W2n, W2s, b2, W3n, W3s, b3):
    src = edge_index[0]
    dst = edge_index[1]
    pad = EPAD - E
    src_p = jnp.concatenate([src, jnp.zeros((pad,), jnp.int32)])
    dst_p = jnp.concatenate([dst, jnp.full((pad,), N, jnp.int32)])
    srcr = src_p.reshape(NW, CH, C)
    dstr = dst_p.reshape(NW, CH, C)
    x_pad = jnp.concatenate(
        [x, jnp.zeros((NPAD - N, D), jnp.float32)], axis=0)

    p1, cnt = _agg_cnt(x_pad, srcr, dstr)
    cnt3 = cnt[:, :, None]
    h1 = _dense(p1, cnt3, x_pad, W1n, W1s, b1, final=False)
    [p2] = _agg(h1, srcr, dstr)
    h2 = _dense(p2, cnt3, h1, W2n, W2s, b2, final=False)
    [p3] = _agg(h2, srcr, dstr)
    out = _dense(p3, cnt3, h2, W3n, W3s, b3, final=True)
    return out[:N]
